# baseline (speedup 1.0000x reference)
import functools

import jax
import jax.numpy as jnp
from jax import lax
from jax.experimental import pallas as pl
from jax.experimental.pallas import tpu as pltpu
from jax.experimental.pallas import tpu_sc as plsc

NC = 2
NS = 16
NW = NC * NS

B = 16384 * 26
D = 64
PER_W = B // NW
NIDX = 128
CHUNK = 512
GPC = CHUNK // NIDX
NCHUNK = PER_W // CHUNK
IDX_ROWS = PER_W // NIDX
NBUF = 3


def _body(x_hbm, idx_hbm, tab_hbm, out_hbm, idx_v, buf0, buf1, buf2,
          xsem, gsem, osem):
    wid = lax.axis_index("s") * NC + lax.axis_index("c")
    base = wid * PER_W
    bufs = (buf0, buf1, buf2)

    pltpu.sync_copy(idx_hbm.at[pl.ds(wid * IDX_ROWS, IDX_ROWS)], idx_v)

    def issue_x(c):
        pltpu.async_copy(
            x_hbm.at[pl.ds(base + c * CHUNK, CHUNK)], bufs[c % NBUF], xsem)

    def issue_gathers(c):
        for j in range(GPC):
            pltpu.async_copy(
                tab_hbm.at[idx_v.at[c * GPC + j]],
                bufs[c % NBUF].at[pl.ds(j * NIDX, NIDX)],
                gsem,
                add=True,
            )

    def issue_out(c):
        pltpu.async_copy(
            bufs[c % NBUF], out_hbm.at[pl.ds(base + c * CHUNK, CHUNK)], osem)

    def wait_x():
        pltpu.make_async_copy(
            x_hbm.at[pl.ds(base, CHUNK)], buf0, xsem).wait()

    def wait_gathers():
        pltpu.make_async_copy(
            x_hbm.at[pl.ds(base, CHUNK)], buf0, gsem).wait()

    def wait_out():
        pltpu.make_async_copy(
            x_hbm.at[pl.ds(base, CHUNK)], buf0, osem).wait()

    issue_x(0)

    for c in range(NCHUNK):
        wait_x()
        issue_gathers(c)
        if c >= 1:
            wait_gathers()
            issue_out(c - 1)
        if c >= 2:
            wait_out()
        if c + 1 < NCHUNK:
            issue_x(c + 1)

    wait_gathers()
    issue_out(NCHUNK - 1)
    wait_out()
    wait_out()


_sc_call = functools.partial(
    pl.kernel,
    mesh=plsc.VectorSubcoreMesh(core_axis_name="c", subcore_axis_name="s"),
    out_type=jax.ShapeDtypeStruct((B, D), jnp.float32),
    scratch_types=[
        pltpu.VMEM((IDX_ROWS, NIDX), jnp.int32),
        pltpu.VMEM((CHUNK, D), jnp.float32),
        pltpu.VMEM((CHUNK, D), jnp.float32),
        pltpu.VMEM((CHUNK, D), jnp.float32),
        pltpu.SemaphoreType.DMA,
        pltpu.SemaphoreType.DMA,
        pltpu.SemaphoreType.DMA,
    ],
    compiler_params=pltpu.CompilerParams(use_tc_tiling_on_sc=False),
)(_body)


@jax.jit
def kernel(x, idxs, embed_weight):
    xf = x.reshape(B, D)
    idxf = idxs.astype(jnp.int32).reshape(B // NIDX, NIDX)
    out = _sc_call(xf, idxf, embed_weight)
    return out.reshape(x.shape)


# speedup vs baseline: 1.0801x; 1.0801x over previous
import functools

import jax
import jax.numpy as jnp
from jax import lax
from jax.experimental import pallas as pl
from jax.experimental.pallas import tpu as pltpu
from jax.experimental.pallas import tpu_sc as plsc

NC = 2
NS = 16
NW = NC * NS

B = 16384 * 26
D = 64
PER_W = B // NW
NIDX = 128
CHUNK = 512
GPC = CHUNK // NIDX
NCHUNK = PER_W // CHUNK
IDX_ROWS = PER_W // NIDX
NBUF = 3


def _body(x_hbm, idx_hbm, tab_hbm, out_hbm, idx_v, buf0, buf1, buf2,
          xsem, gsem, osem):
    wid = lax.axis_index("s") * NC + lax.axis_index("c")
    base = wid * PER_W
    bufs = (buf0, buf1, buf2)

    pltpu.sync_copy(idx_hbm.at[pl.ds(wid * IDX_ROWS, IDX_ROWS)], idx_v)

    def issue_x(c):
        pltpu.async_copy(
            x_hbm.at[pl.ds(base + c * CHUNK, CHUNK)], bufs[c % NBUF], xsem)

    def issue_gathers(c):
        for j in range(GPC):
            pltpu.async_copy(
                tab_hbm.at[idx_v.at[c * GPC + j]],
                bufs[c % NBUF].at[pl.ds(j * NIDX, NIDX)],
                gsem,
                add=True,
            )

    def issue_out(c):
        pltpu.async_copy(
            bufs[c % NBUF], out_hbm.at[pl.ds(base + c * CHUNK, CHUNK)], osem)

    def wait_x():
        pltpu.make_async_copy(
            x_hbm.at[pl.ds(base, CHUNK)], buf0, xsem).wait()

    def wait_gathers():
        pltpu.make_async_copy(
            x_hbm.at[pl.ds(base, CHUNK)], buf0, gsem).wait()

    def wait_out():
        pltpu.make_async_copy(
            x_hbm.at[pl.ds(base, CHUNK)], buf0, osem).wait()

    issue_x(0)
    issue_x(1)

    for c in range(NCHUNK):
        wait_x()
        issue_gathers(c)
        wait_gathers()
        issue_out(c)
        if c >= 1:
            wait_out()
        if c + 2 < NCHUNK:
            issue_x(c + 2)

    wait_out()


_sc_call = functools.partial(
    pl.kernel,
    mesh=plsc.VectorSubcoreMesh(core_axis_name="c", subcore_axis_name="s"),
    out_type=jax.ShapeDtypeStruct((B, D), jnp.float32),
    scratch_types=[
        pltpu.VMEM((IDX_ROWS, NIDX), jnp.int32),
        pltpu.VMEM((CHUNK, D), jnp.float32),
        pltpu.VMEM((CHUNK, D), jnp.float32),
        pltpu.VMEM((CHUNK, D), jnp.float32),
        pltpu.SemaphoreType.DMA,
        pltpu.SemaphoreType.DMA,
        pltpu.SemaphoreType.DMA,
    ],
    compiler_params=pltpu.CompilerParams(use_tc_tiling_on_sc=False),
)(_body)


@jax.jit
def kernel(x, idxs, embed_weight):
    xf = x.reshape(B, D)
    idxf = idxs.astype(jnp.int32).reshape(B // NIDX, NIDX)
    out = _sc_call(xf, idxf, embed_weight)
    return out.reshape(x.shape)
